# baseline (speedup 1.0000x reference)
import functools

import jax
import jax.numpy as jnp
from jax import lax
from jax.experimental import pallas as pl
from jax.experimental.pallas import tpu as pltpu
from jax.experimental.pallas import tpu_sc as plsc

_NC = 2
_NS = 16
_NW = _NC * _NS
_D = 128
_EC = 128
_R0 = 48
_R1 = 112


def _sc_mesh():
    return plsc.VectorSubcoreMesh(core_axis_name="c", subcore_axis_name="s")


def _sc_degree(dst2, n_pad):
    rows_per_tile = dst2.shape[0] // _NW
    npt = n_pad // _NS

    @functools.partial(
        pl.kernel,
        out_type=[
            jax.ShapeDtypeStruct((n_pad,), jnp.float32),
            jax.ShapeDtypeStruct((n_pad,), jnp.float32),
        ],
        mesh=_sc_mesh(),
        scratch_types=[
            pltpu.VMEM((rows_per_tile, _EC), jnp.int32),
            pltpu.VMEM((_EC,), jnp.float32),
            pltpu.VMEM((npt,), jnp.float32),
            pltpu.VMEM_SHARED((n_pad,), jnp.float32),
            pltpu.SemaphoreType.DMA,
        ],
    )
    def k(dst_hbm, out0, out1, dst_v, ones_v, stage_v, deg_sh, sem):
        c = lax.axis_index("c")
        s = lax.axis_index("s")
        w = c * _NS + s

        def fill_ones(i, _):
            ones_v[pl.ds(i * 16, 16)] = jnp.ones((16,), jnp.float32)
            return 0

        lax.fori_loop(0, _EC // 16, fill_ones, 0)

        def fill_zero(i, _):
            stage_v[pl.ds(i * 16, 16)] = jnp.zeros((16,), jnp.float32)
            return 0

        lax.fori_loop(0, npt // 16, fill_zero, 0)
        pltpu.sync_copy(stage_v, deg_sh.at[pl.ds(s * npt, npt)])
        pltpu.async_copy(
            dst_hbm.at[pl.ds(w * rows_per_tile, rows_per_tile)], dst_v, sem
        ).wait()
        plsc.subcore_barrier()

        def body(k, _):
            for j in range(4):
                pltpu.async_copy(
                    ones_v, deg_sh.at[dst_v.at[k * 4 + j]], sem, add=True
                )
            for j in range(4):
                pltpu.make_async_copy(
                    ones_v, deg_sh.at[pl.ds(0, _EC)], sem
                ).wait()
            return 0

        lax.fori_loop(0, rows_per_tile // 4, body, 0)
        plsc.subcore_barrier()
        pltpu.sync_copy(deg_sh.at[pl.ds(s * npt, npt)], stage_v)

        @pl.when(c == 0)
        def _():
            pltpu.sync_copy(stage_v, out0.at[pl.ds(s * npt, npt)])

        @pl.when(c == 1)
        def _():
            pltpu.sync_copy(stage_v, out1.at[pl.ds(s * npt, npt)])

    return k(dst2)


def _sc_propagate(u, src2, dst2, n_pad, r0, r1):
    npt = n_pad // _NS
    rmax = max(r0, r1)

    @functools.partial(
        pl.kernel,
        out_type=jax.ShapeDtypeStruct((_NC, n_pad, _D), jnp.float32),
        mesh=_sc_mesh(),
        scratch_types=[
            pltpu.VMEM((rmax, _EC), jnp.int32),
            pltpu.VMEM((rmax, _EC), jnp.int32),
            pltpu.VMEM_SHARED((n_pad, _D), jnp.float32),
            pltpu.VMEM((_EC, _D), jnp.float32),
            pltpu.SemaphoreType.DMA,
            pltpu.SemaphoreType.DMA,
        ],
    )
    def k(u_hbm, src_hbm, dst_hbm, out_hbm, src_v, dst_v, y_sh, rows_v,
          gsem, sem):
        c = lax.axis_index("c")
        s = lax.axis_index("s")
        rpt = jnp.where(c == 0, r0, r1)
        base_row = c * _NS * r0 + s * rpt

        def zrow(i, _):
            for kk in range(_D // 16):
                rows_v[i, pl.ds(kk * 16, 16)] = jnp.zeros((16,), jnp.float32)
            return 0

        lax.fori_loop(0, _EC, zrow, 0)

        def zsh(j, _):
            pltpu.sync_copy(rows_v, y_sh.at[pl.ds(s * npt + j * _EC, _EC)])
            return 0

        lax.fori_loop(0, npt // _EC, zsh, 0)
        pltpu.async_copy(src_hbm.at[pl.ds(base_row, rmax)], src_v, sem).wait()
        pltpu.async_copy(dst_hbm.at[pl.ds(base_row, rmax)], dst_v, sem).wait()
        plsc.subcore_barrier()

        pltpu.async_copy(u_hbm.at[src_v.at[0]], rows_v, gsem)

        def body(g, _):
            pltpu.make_async_copy(
                u_hbm.at[pl.ds(0, _EC)], rows_v, gsem
            ).wait()
            pltpu.sync_copy(rows_v, y_sh.at[dst_v.at[g]], add=True)

            @pl.when(g < rpt - 1)
            def _():
                pltpu.async_copy(u_hbm.at[src_v.at[g + 1]], rows_v, gsem)

            return 0

        lax.fori_loop(0, rpt, body, 0)
        plsc.subcore_barrier()

        pltpu.sync_copy(
            y_sh.at[pl.ds(s * npt, npt)], out_hbm.at[c, pl.ds(s * npt, npt)]
        )

    return k(u, src2, dst2)


def _sc_take(full, sr3):
    @functools.partial(
        pl.kernel,
        out_type=jax.ShapeDtypeStruct((_NW * 256, _D), jnp.float32),
        mesh=_sc_mesh(),
        scratch_types=[
            pltpu.VMEM((2, 128), jnp.int32),
            pltpu.VMEM((128, _D), jnp.float32),
            pltpu.SemaphoreType.DMA,
        ],
    )
    def k(full_hbm, sr_hbm, out_hbm, idx_v, rows_v, sem):
        c = lax.axis_index("c")
        s = lax.axis_index("s")
        w = c * _NS + s
        pltpu.async_copy(sr_hbm.at[w], idx_v, sem).wait()
        for j in range(2):
            pltpu.async_copy(full_hbm.at[idx_v.at[j]], rows_v, sem).wait()
            pltpu.sync_copy(rows_v, out_hbm.at[pl.ds(w * 256 + j * 128, 128)])

    return k(full, sr3)


def _tc_dis(d0, d1):
    def body(a_ref, b_ref, o_ref):
        deg = a_ref[...] + b_ref[...]
        o_ref[...] = jnp.where(deg > 0.0, lax.rsqrt(jnp.maximum(deg, 1.0)), 0.0)

    return pl.pallas_call(
        body, out_shape=jax.ShapeDtypeStruct(d0.shape, jnp.float32)
    )(d0, d1)


def _tc_prolog(emb_pad, dis):
    n_pad = emb_pad.shape[0]
    blk = n_pad // 8

    def body(emb_ref, dis_ref, u0_ref, z0_ref):
        x = emb_ref[...]
        u0_ref[...] = x * dis_ref[...]
        nrm = jnp.sqrt(jnp.sum(x * x, axis=1, keepdims=True))
        z0_ref[...] = x / jnp.maximum(nrm, 1e-12)

    return pl.pallas_call(
        body,
        grid=(8,),
        in_specs=[
            pl.BlockSpec((blk, _D), lambda i: (i, 0)),
            pl.BlockSpec((blk, 1), lambda i: (i, 0)),
        ],
        out_specs=[
            pl.BlockSpec((blk, _D), lambda i: (i, 0)),
            pl.BlockSpec((blk, _D), lambda i: (i, 0)),
        ],
        out_shape=[
            jax.ShapeDtypeStruct((n_pad, _D), jnp.float32),
            jax.ShapeDtypeStruct((n_pad, _D), jnp.float32),
        ],
    )(emb_pad, dis)


def _tc_mid(part, dis, zsum):
    n_pad = dis.shape[0]
    blk = n_pad // 8

    def body(part_ref, dis_ref, zsum_ref, u_ref, zout_ref):
        x = (part_ref[0] + part_ref[1]) * dis_ref[...]
        nrm = jnp.sqrt(jnp.sum(x * x, axis=1, keepdims=True))
        zout_ref[...] = zsum_ref[...] + x / jnp.maximum(nrm, 1e-12)
        u_ref[...] = x * dis_ref[...]

    return pl.pallas_call(
        body,
        grid=(8,),
        in_specs=[
            pl.BlockSpec((_NC, blk, _D), lambda i: (0, i, 0)),
            pl.BlockSpec((blk, 1), lambda i: (i, 0)),
            pl.BlockSpec((blk, _D), lambda i: (i, 0)),
        ],
        out_specs=[
            pl.BlockSpec((blk, _D), lambda i: (i, 0)),
            pl.BlockSpec((blk, _D), lambda i: (i, 0)),
        ],
        out_shape=[
            jax.ShapeDtypeStruct((n_pad, _D), jnp.float32),
            jax.ShapeDtypeStruct((n_pad, _D), jnp.float32),
        ],
    )(part, dis, zsum)


def _tc_final(part, dis, zsum, W, b2):
    n_pad = dis.shape[0]
    blk = n_pad // 8

    def body(part_ref, dis_ref, zsum_ref, w_ref, b_ref, out_ref):
        x = (part_ref[0] + part_ref[1]) * dis_ref[...]
        nrm = jnp.sqrt(jnp.sum(x * x, axis=1, keepdims=True))
        zm = (zsum_ref[...] + x / jnp.maximum(nrm, 1e-12)) * 0.25
        out_ref[...] = (
            lax.dot_general(
                zm,
                w_ref[...],
                (((1,), (1,)), ((), ())),
                preferred_element_type=jnp.float32,
            )
            + b_ref[...]
        )

    return pl.pallas_call(
        body,
        grid=(8,),
        in_specs=[
            pl.BlockSpec((_NC, blk, _D), lambda i: (0, i, 0)),
            pl.BlockSpec((blk, 1), lambda i: (i, 0)),
            pl.BlockSpec((blk, _D), lambda i: (i, 0)),
            pl.BlockSpec((_D, _D), lambda i: (0, 0)),
            pl.BlockSpec((1, _D), lambda i: (0, 0)),
        ],
        out_specs=pl.BlockSpec((blk, _D), lambda i: (i, 0)),
        out_shape=jax.ShapeDtypeStruct((n_pad, _D), jnp.float32),
    )(part, dis, zsum, W, b2)


def kernel(senders, receivers, emb, edge_index, W, b):
    n = emb.shape[0]
    n_pad = -(-n // 1280) * 1280
    e_rows = edge_index.shape[1] // _EC
    assert _NS * (_R0 + _R1) >= e_rows and max(_R0, _R1) % 8 == 0
    e_rows_pad = -(-(_NS * (_R0 + _R1) + max(_R0, _R1)) // 256) * 256
    src2 = edge_index[0].astype(jnp.int32).reshape(e_rows, _EC)
    dst2 = edge_index[1].astype(jnp.int32).reshape(e_rows, _EC)
    pad_rows = ((0, e_rows_pad - e_rows), (0, 0))
    src2 = jnp.pad(src2, pad_rows, constant_values=n)
    dst2 = jnp.pad(dst2, pad_rows, constant_values=n)
    emb_pad = jnp.pad(emb, ((0, n_pad - n), (0, 0)))

    deg0, deg1 = _sc_degree(dst2, n_pad)
    dis2d = _tc_dis(deg0.reshape(-1, 128), deg1.reshape(-1, 128))
    dis = dis2d.reshape(n_pad, 1)
    u, zsum = _tc_prolog(emb_pad, dis)
    for layer in range(3):
        part = _sc_propagate(u, src2, dst2, n_pad, _R0, _R1)
        if layer < 2:
            u, zsum = _tc_mid(part, dis, zsum)
        else:
            full = _tc_final(part, dis, zsum, W, b.reshape(1, _D))

    sr3 = (
        jnp.concatenate([senders, receivers])
        .astype(jnp.int32)
        .reshape(_NW, 2, 128)
    )
    both = _sc_take(full, sr3)
    nb = senders.shape[0]
    return both[:nb], both[nb:]


# speedup vs baseline: 1.2413x; 1.2413x over previous
import functools

import jax
import jax.numpy as jnp
from jax import lax
from jax.experimental import pallas as pl
from jax.experimental.pallas import tpu as pltpu
from jax.experimental.pallas import tpu_sc as plsc

_NC = 2
_NS = 16
_NW = _NC * _NS
_D = 128
_EC = 128
_R0 = 112
_R1 = 48


def _sc_mesh():
    return plsc.VectorSubcoreMesh(core_axis_name="c", subcore_axis_name="s")


def _sc_degree(dst2, n_pad):
    rows_per_tile = dst2.shape[0] // _NW
    npt = n_pad // _NS

    @functools.partial(
        pl.kernel,
        out_type=[
            jax.ShapeDtypeStruct((n_pad,), jnp.float32),
            jax.ShapeDtypeStruct((n_pad,), jnp.float32),
        ],
        mesh=_sc_mesh(),
        scratch_types=[
            pltpu.VMEM((rows_per_tile, _EC), jnp.int32),
            pltpu.VMEM((_EC,), jnp.float32),
            pltpu.VMEM((npt,), jnp.float32),
            pltpu.VMEM_SHARED((n_pad,), jnp.float32),
            pltpu.SemaphoreType.DMA,
        ],
    )
    def k(dst_hbm, out0, out1, dst_v, ones_v, stage_v, deg_sh, sem):
        c = lax.axis_index("c")
        s = lax.axis_index("s")
        w = c * _NS + s

        def fill_ones(i, _):
            ones_v[pl.ds(i * 16, 16)] = jnp.ones((16,), jnp.float32)
            return 0

        lax.fori_loop(0, _EC // 16, fill_ones, 0)

        def fill_zero(i, _):
            stage_v[pl.ds(i * 16, 16)] = jnp.zeros((16,), jnp.float32)
            return 0

        lax.fori_loop(0, npt // 16, fill_zero, 0)
        pltpu.sync_copy(stage_v, deg_sh.at[pl.ds(s * npt, npt)])
        pltpu.async_copy(
            dst_hbm.at[pl.ds(w * rows_per_tile, rows_per_tile)], dst_v, sem
        ).wait()
        plsc.subcore_barrier()

        def body(k, _):
            for j in range(4):
                pltpu.async_copy(
                    ones_v, deg_sh.at[dst_v.at[k * 4 + j]], sem, add=True
                )
            for j in range(4):
                pltpu.make_async_copy(
                    ones_v, deg_sh.at[pl.ds(0, _EC)], sem
                ).wait()
            return 0

        lax.fori_loop(0, rows_per_tile // 4, body, 0)
        plsc.subcore_barrier()
        pltpu.sync_copy(deg_sh.at[pl.ds(s * npt, npt)], stage_v)

        @pl.when(c == 0)
        def _():
            pltpu.sync_copy(stage_v, out0.at[pl.ds(s * npt, npt)])

        @pl.when(c == 1)
        def _():
            pltpu.sync_copy(stage_v, out1.at[pl.ds(s * npt, npt)])

    return k(dst2)


def _sc_propagate(u, src2, dst2, n_pad, r0, r1):
    npt = n_pad // _NS
    rmax = max(r0, r1)

    @functools.partial(
        pl.kernel,
        out_type=jax.ShapeDtypeStruct((_NC, n_pad, _D), jnp.float32),
        mesh=_sc_mesh(),
        scratch_types=[
            pltpu.VMEM((rmax, _EC), jnp.int32),
            pltpu.VMEM((rmax, _EC), jnp.int32),
            pltpu.VMEM_SHARED((n_pad, _D), jnp.float32),
            pltpu.VMEM((_EC, _D), jnp.float32),
            pltpu.SemaphoreType.DMA,
            pltpu.SemaphoreType.DMA,
        ],
    )
    def k(u_hbm, src_hbm, dst_hbm, out_hbm, src_v, dst_v, y_sh, rows_v,
          gsem, sem):
        c = lax.axis_index("c")
        s = lax.axis_index("s")
        rpt = jnp.where(c == 0, r0, r1)
        base_row = c * _NS * r0 + s * rpt

        def zrow(i, _):
            for kk in range(_D // 16):
                rows_v[i, pl.ds(kk * 16, 16)] = jnp.zeros((16,), jnp.float32)
            return 0

        lax.fori_loop(0, _EC, zrow, 0)

        def zsh(j, _):
            pltpu.sync_copy(rows_v, y_sh.at[pl.ds(s * npt + j * _EC, _EC)])
            return 0

        lax.fori_loop(0, npt // _EC, zsh, 0)
        pltpu.async_copy(src_hbm.at[pl.ds(base_row, rmax)], src_v, sem).wait()
        pltpu.async_copy(dst_hbm.at[pl.ds(base_row, rmax)], dst_v, sem).wait()
        plsc.subcore_barrier()

        pltpu.async_copy(u_hbm.at[src_v.at[0]], rows_v, gsem)

        def body(g, _):
            pltpu.make_async_copy(
                u_hbm.at[pl.ds(0, _EC)], rows_v, gsem
            ).wait()
            pltpu.sync_copy(rows_v, y_sh.at[dst_v.at[g]], add=True)

            @pl.when(g < rpt - 1)
            def _():
                pltpu.async_copy(u_hbm.at[src_v.at[g + 1]], rows_v, gsem)

            return 0

        lax.fori_loop(0, rpt, body, 0)
        plsc.subcore_barrier()

        pltpu.sync_copy(
            y_sh.at[pl.ds(s * npt, npt)], out_hbm.at[c, pl.ds(s * npt, npt)]
        )

    return k(u, src2, dst2)


def _sc_take(full, sr3):
    @functools.partial(
        pl.kernel,
        out_type=jax.ShapeDtypeStruct((_NW * 256, _D), jnp.float32),
        mesh=_sc_mesh(),
        scratch_types=[
            pltpu.VMEM((2, 128), jnp.int32),
            pltpu.VMEM((128, _D), jnp.float32),
            pltpu.SemaphoreType.DMA,
        ],
    )
    def k(full_hbm, sr_hbm, out_hbm, idx_v, rows_v, sem):
        c = lax.axis_index("c")
        s = lax.axis_index("s")
        w = c * _NS + s
        pltpu.async_copy(sr_hbm.at[w], idx_v, sem).wait()
        for j in range(2):
            pltpu.async_copy(full_hbm.at[idx_v.at[j]], rows_v, sem).wait()
            pltpu.sync_copy(rows_v, out_hbm.at[pl.ds(w * 256 + j * 128, 128)])

    return k(full, sr3)


def _tc_dis(d0, d1):
    def body(a_ref, b_ref, o_ref):
        deg = a_ref[...] + b_ref[...]
        o_ref[...] = jnp.where(deg > 0.0, lax.rsqrt(jnp.maximum(deg, 1.0)), 0.0)

    return pl.pallas_call(
        body, out_shape=jax.ShapeDtypeStruct(d0.shape, jnp.float32)
    )(d0, d1)


def _tc_prolog(emb_pad, dis):
    n_pad = emb_pad.shape[0]
    blk = n_pad // 8

    def body(emb_ref, dis_ref, u0_ref, z0_ref):
        x = emb_ref[...]
        u0_ref[...] = x * dis_ref[...]
        nrm = jnp.sqrt(jnp.sum(x * x, axis=1, keepdims=True))
        z0_ref[...] = x / jnp.maximum(nrm, 1e-12)

    return pl.pallas_call(
        body,
        grid=(8,),
        in_specs=[
            pl.BlockSpec((blk, _D), lambda i: (i, 0)),
            pl.BlockSpec((blk, 1), lambda i: (i, 0)),
        ],
        out_specs=[
            pl.BlockSpec((blk, _D), lambda i: (i, 0)),
            pl.BlockSpec((blk, _D), lambda i: (i, 0)),
        ],
        out_shape=[
            jax.ShapeDtypeStruct((n_pad, _D), jnp.float32),
            jax.ShapeDtypeStruct((n_pad, _D), jnp.float32),
        ],
    )(emb_pad, dis)


def _tc_mid(part, dis, zsum):
    n_pad = dis.shape[0]
    blk = n_pad // 8

    def body(part_ref, dis_ref, zsum_ref, u_ref, zout_ref):
        x = (part_ref[0] + part_ref[1]) * dis_ref[...]
        nrm = jnp.sqrt(jnp.sum(x * x, axis=1, keepdims=True))
        zout_ref[...] = zsum_ref[...] + x / jnp.maximum(nrm, 1e-12)
        u_ref[...] = x * dis_ref[...]

    return pl.pallas_call(
        body,
        grid=(8,),
        in_specs=[
            pl.BlockSpec((_NC, blk, _D), lambda i: (0, i, 0)),
            pl.BlockSpec((blk, 1), lambda i: (i, 0)),
            pl.BlockSpec((blk, _D), lambda i: (i, 0)),
        ],
        out_specs=[
            pl.BlockSpec((blk, _D), lambda i: (i, 0)),
            pl.BlockSpec((blk, _D), lambda i: (i, 0)),
        ],
        out_shape=[
            jax.ShapeDtypeStruct((n_pad, _D), jnp.float32),
            jax.ShapeDtypeStruct((n_pad, _D), jnp.float32),
        ],
    )(part, dis, zsum)


def _tc_final(part, dis, zsum, W, b2):
    n_pad = dis.shape[0]
    blk = n_pad // 8

    def body(part_ref, dis_ref, zsum_ref, w_ref, b_ref, out_ref):
        x = (part_ref[0] + part_ref[1]) * dis_ref[...]
        nrm = jnp.sqrt(jnp.sum(x * x, axis=1, keepdims=True))
        zm = (zsum_ref[...] + x / jnp.maximum(nrm, 1e-12)) * 0.25
        out_ref[...] = (
            lax.dot_general(
                zm,
                w_ref[...],
                (((1,), (1,)), ((), ())),
                preferred_element_type=jnp.float32,
            )
            + b_ref[...]
        )

    return pl.pallas_call(
        body,
        grid=(8,),
        in_specs=[
            pl.BlockSpec((_NC, blk, _D), lambda i: (0, i, 0)),
            pl.BlockSpec((blk, 1), lambda i: (i, 0)),
            pl.BlockSpec((blk, _D), lambda i: (i, 0)),
            pl.BlockSpec((_D, _D), lambda i: (0, 0)),
            pl.BlockSpec((1, _D), lambda i: (0, 0)),
        ],
        out_specs=pl.BlockSpec((blk, _D), lambda i: (i, 0)),
        out_shape=jax.ShapeDtypeStruct((n_pad, _D), jnp.float32),
    )(part, dis, zsum, W, b2)


def kernel(senders, receivers, emb, edge_index, W, b):
    n = emb.shape[0]
    n_pad = -(-n // 1280) * 1280
    e_rows = edge_index.shape[1] // _EC
    assert _NS * (_R0 + _R1) >= e_rows and max(_R0, _R1) % 8 == 0
    e_rows_pad = -(-(_NS * (_R0 + _R1) + max(_R0, _R1)) // 256) * 256
    src2 = edge_index[0].astype(jnp.int32).reshape(e_rows, _EC)
    dst2 = edge_index[1].astype(jnp.int32).reshape(e_rows, _EC)
    pad_rows = ((0, e_rows_pad - e_rows), (0, 0))
    src2 = jnp.pad(src2, pad_rows, constant_values=n)
    dst2 = jnp.pad(dst2, pad_rows, constant_values=n)
    emb_pad = jnp.pad(emb, ((0, n_pad - n), (0, 0)))

    deg0, deg1 = _sc_degree(dst2, n_pad)
    dis2d = _tc_dis(deg0.reshape(-1, 128), deg1.reshape(-1, 128))
    dis = dis2d.reshape(n_pad, 1)
    u, zsum = _tc_prolog(emb_pad, dis)
    for layer in range(3):
        part = _sc_propagate(u, src2, dst2, n_pad, _R0, _R1)
        if layer < 2:
            u, zsum = _tc_mid(part, dis, zsum)
        else:
            full = _tc_final(part, dis, zsum, W, b.reshape(1, _D))

    sr3 = (
        jnp.concatenate([senders, receivers])
        .astype(jnp.int32)
        .reshape(_NW, 2, 128)
    )
    both = _sc_take(full, sr3)
    nb = senders.shape[0]
    return both[:nb], both[nb:]


# speedup vs baseline: 1.2844x; 1.0347x over previous
import functools

import jax
import jax.numpy as jnp
from jax import lax
from jax.experimental import pallas as pl
from jax.experimental.pallas import tpu as pltpu
from jax.experimental.pallas import tpu_sc as plsc

_NC = 2
_NS = 16
_NW = _NC * _NS
_D = 128
_EC = 128
_R0 = 120
_R1 = 40


def _sc_mesh():
    return plsc.VectorSubcoreMesh(core_axis_name="c", subcore_axis_name="s")


def _sc_degree(dst2, n_pad):
    rows_per_tile = dst2.shape[0] // _NW
    npt = n_pad // _NS

    @functools.partial(
        pl.kernel,
        out_type=[
            jax.ShapeDtypeStruct((n_pad,), jnp.float32),
            jax.ShapeDtypeStruct((n_pad,), jnp.float32),
        ],
        mesh=_sc_mesh(),
        scratch_types=[
            pltpu.VMEM((rows_per_tile, _EC), jnp.int32),
            pltpu.VMEM((_EC,), jnp.float32),
            pltpu.VMEM((npt,), jnp.float32),
            pltpu.VMEM_SHARED((n_pad,), jnp.float32),
            pltpu.SemaphoreType.DMA,
        ],
    )
    def k(dst_hbm, out0, out1, dst_v, ones_v, stage_v, deg_sh, sem):
        c = lax.axis_index("c")
        s = lax.axis_index("s")
        w = c * _NS + s

        def fill_ones(i, _):
            ones_v[pl.ds(i * 16, 16)] = jnp.ones((16,), jnp.float32)
            return 0

        lax.fori_loop(0, _EC // 16, fill_ones, 0)

        def fill_zero(i, _):
            stage_v[pl.ds(i * 16, 16)] = jnp.zeros((16,), jnp.float32)
            return 0

        lax.fori_loop(0, npt // 16, fill_zero, 0)
        pltpu.sync_copy(stage_v, deg_sh.at[pl.ds(s * npt, npt)])
        pltpu.async_copy(
            dst_hbm.at[pl.ds(w * rows_per_tile, rows_per_tile)], dst_v, sem
        ).wait()
        plsc.subcore_barrier()

        def body(k, _):
            for j in range(4):
                pltpu.async_copy(
                    ones_v, deg_sh.at[dst_v.at[k * 4 + j]], sem, add=True
                )
            for j in range(4):
                pltpu.make_async_copy(
                    ones_v, deg_sh.at[pl.ds(0, _EC)], sem
                ).wait()
            return 0

        lax.fori_loop(0, rows_per_tile // 4, body, 0)
        plsc.subcore_barrier()
        pltpu.sync_copy(deg_sh.at[pl.ds(s * npt, npt)], stage_v)

        @pl.when(c == 0)
        def _():
            pltpu.sync_copy(stage_v, out0.at[pl.ds(s * npt, npt)])

        @pl.when(c == 1)
        def _():
            pltpu.sync_copy(stage_v, out1.at[pl.ds(s * npt, npt)])

    return k(dst2)


def _sc_propagate(u, src2, dst2, n_pad, r0, r1):
    npt = n_pad // _NS
    rmax = max(r0, r1)

    @functools.partial(
        pl.kernel,
        out_type=jax.ShapeDtypeStruct((_NC, n_pad, _D), jnp.float32),
        mesh=_sc_mesh(),
        scratch_types=[
            pltpu.VMEM((rmax, _EC), jnp.int32),
            pltpu.VMEM((rmax, _EC), jnp.int32),
            pltpu.VMEM_SHARED((n_pad, _D), jnp.float32),
            pltpu.VMEM((_EC, _D), jnp.float32),
            pltpu.SemaphoreType.DMA,
            pltpu.SemaphoreType.DMA,
        ],
    )
    def k(u_hbm, src_hbm, dst_hbm, out_hbm, src_v, dst_v, y_sh, rows_v,
          gsem, sem):
        c = lax.axis_index("c")
        s = lax.axis_index("s")
        rpt = jnp.where(c == 0, r0, r1)
        base_row = c * _NS * r0 + s * rpt

        def zrow(i, _):
            for kk in range(_D // 16):
                rows_v[i, pl.ds(kk * 16, 16)] = jnp.zeros((16,), jnp.float32)
            return 0

        lax.fori_loop(0, _EC, zrow, 0)

        def zsh(j, _):
            pltpu.sync_copy(rows_v, y_sh.at[pl.ds(s * npt + j * _EC, _EC)])
            return 0

        lax.fori_loop(0, npt // _EC, zsh, 0)
        pltpu.async_copy(src_hbm.at[pl.ds(base_row, rmax)], src_v, sem).wait()
        pltpu.async_copy(dst_hbm.at[pl.ds(base_row, rmax)], dst_v, sem).wait()
        plsc.subcore_barrier()

        pltpu.async_copy(u_hbm.at[src_v.at[0]], rows_v, gsem)

        def body(g, _):
            pltpu.make_async_copy(
                u_hbm.at[pl.ds(0, _EC)], rows_v, gsem
            ).wait()
            pltpu.sync_copy(rows_v, y_sh.at[dst_v.at[g]], add=True)

            @pl.when(g < rpt - 1)
            def _():
                pltpu.async_copy(u_hbm.at[src_v.at[g + 1]], rows_v, gsem)

            return 0

        lax.fori_loop(0, rpt, body, 0)
        plsc.subcore_barrier()

        pltpu.sync_copy(
            y_sh.at[pl.ds(s * npt, npt)], out_hbm.at[c, pl.ds(s * npt, npt)]
        )

    return k(u, src2, dst2)


def _sc_take(full, sr3):
    @functools.partial(
        pl.kernel,
        out_type=jax.ShapeDtypeStruct((_NW * 256, _D), jnp.float32),
        mesh=_sc_mesh(),
        scratch_types=[
            pltpu.VMEM((2, 128), jnp.int32),
            pltpu.VMEM((128, _D), jnp.float32),
            pltpu.SemaphoreType.DMA,
        ],
    )
    def k(full_hbm, sr_hbm, out_hbm, idx_v, rows_v, sem):
        c = lax.axis_index("c")
        s = lax.axis_index("s")
        w = c * _NS + s
        pltpu.async_copy(sr_hbm.at[w], idx_v, sem).wait()
        for j in range(2):
            pltpu.async_copy(full_hbm.at[idx_v.at[j]], rows_v, sem).wait()
            pltpu.sync_copy(rows_v, out_hbm.at[pl.ds(w * 256 + j * 128, 128)])

    return k(full, sr3)


def _tc_dis(d0, d1):
    def body(a_ref, b_ref, o_ref):
        deg = a_ref[...] + b_ref[...]
        o_ref[...] = jnp.where(deg > 0.0, lax.rsqrt(jnp.maximum(deg, 1.0)), 0.0)

    return pl.pallas_call(
        body, out_shape=jax.ShapeDtypeStruct(d0.shape, jnp.float32)
    )(d0, d1)


def _tc_prolog(emb_pad, dis):
    n_pad = emb_pad.shape[0]
    blk = n_pad // 8

    def body(emb_ref, dis_ref, u0_ref, z0_ref):
        x = emb_ref[...]
        u0_ref[...] = x * dis_ref[...]
        nrm = jnp.sqrt(jnp.sum(x * x, axis=1, keepdims=True))
        z0_ref[...] = x / jnp.maximum(nrm, 1e-12)

    return pl.pallas_call(
        body,
        grid=(8,),
        in_specs=[
            pl.BlockSpec((blk, _D), lambda i: (i, 0)),
            pl.BlockSpec((blk, 1), lambda i: (i, 0)),
        ],
        out_specs=[
            pl.BlockSpec((blk, _D), lambda i: (i, 0)),
            pl.BlockSpec((blk, _D), lambda i: (i, 0)),
        ],
        out_shape=[
            jax.ShapeDtypeStruct((n_pad, _D), jnp.float32),
            jax.ShapeDtypeStruct((n_pad, _D), jnp.float32),
        ],
    )(emb_pad, dis)


def _tc_mid(part, dis, zsum):
    n_pad = dis.shape[0]
    blk = n_pad // 8

    def body(part_ref, dis_ref, zsum_ref, u_ref, zout_ref):
        x = (part_ref[0] + part_ref[1]) * dis_ref[...]
        nrm = jnp.sqrt(jnp.sum(x * x, axis=1, keepdims=True))
        zout_ref[...] = zsum_ref[...] + x / jnp.maximum(nrm, 1e-12)
        u_ref[...] = x * dis_ref[...]

    return pl.pallas_call(
        body,
        grid=(8,),
        in_specs=[
            pl.BlockSpec((_NC, blk, _D), lambda i: (0, i, 0)),
            pl.BlockSpec((blk, 1), lambda i: (i, 0)),
            pl.BlockSpec((blk, _D), lambda i: (i, 0)),
        ],
        out_specs=[
            pl.BlockSpec((blk, _D), lambda i: (i, 0)),
            pl.BlockSpec((blk, _D), lambda i: (i, 0)),
        ],
        out_shape=[
            jax.ShapeDtypeStruct((n_pad, _D), jnp.float32),
            jax.ShapeDtypeStruct((n_pad, _D), jnp.float32),
        ],
    )(part, dis, zsum)


def _tc_final(part, dis, zsum, W, b2):
    n_pad = dis.shape[0]
    blk = n_pad // 8

    def body(part_ref, dis_ref, zsum_ref, w_ref, b_ref, out_ref):
        x = (part_ref[0] + part_ref[1]) * dis_ref[...]
        nrm = jnp.sqrt(jnp.sum(x * x, axis=1, keepdims=True))
        zm = (zsum_ref[...] + x / jnp.maximum(nrm, 1e-12)) * 0.25
        out_ref[...] = (
            lax.dot_general(
                zm,
                w_ref[...],
                (((1,), (1,)), ((), ())),
                preferred_element_type=jnp.float32,
            )
            + b_ref[...]
        )

    return pl.pallas_call(
        body,
        grid=(8,),
        in_specs=[
            pl.BlockSpec((_NC, blk, _D), lambda i: (0, i, 0)),
            pl.BlockSpec((blk, 1), lambda i: (i, 0)),
            pl.BlockSpec((blk, _D), lambda i: (i, 0)),
            pl.BlockSpec((_D, _D), lambda i: (0, 0)),
            pl.BlockSpec((1, _D), lambda i: (0, 0)),
        ],
        out_specs=pl.BlockSpec((blk, _D), lambda i: (i, 0)),
        out_shape=jax.ShapeDtypeStruct((n_pad, _D), jnp.float32),
    )(part, dis, zsum, W, b2)


def kernel(senders, receivers, emb, edge_index, W, b):
    n = emb.shape[0]
    n_pad = -(-n // 1280) * 1280
    e_rows = edge_index.shape[1] // _EC
    assert _NS * (_R0 + _R1) >= e_rows and max(_R0, _R1) % 8 == 0
    e_rows_pad = -(-(_NS * (_R0 + _R1) + max(_R0, _R1)) // 256) * 256
    src2 = edge_index[0].astype(jnp.int32).reshape(e_rows, _EC)
    dst2 = edge_index[1].astype(jnp.int32).reshape(e_rows, _EC)
    pad_rows = ((0, e_rows_pad - e_rows), (0, 0))
    src2 = jnp.pad(src2, pad_rows, constant_values=n)
    dst2 = jnp.pad(dst2, pad_rows, constant_values=n)
    emb_pad = jnp.pad(emb, ((0, n_pad - n), (0, 0)))

    deg0, deg1 = _sc_degree(dst2, n_pad)
    dis2d = _tc_dis(deg0.reshape(-1, 128), deg1.reshape(-1, 128))
    dis = dis2d.reshape(n_pad, 1)
    u, zsum = _tc_prolog(emb_pad, dis)
    for layer in range(3):
        part = _sc_propagate(u, src2, dst2, n_pad, _R0, _R1)
        if layer < 2:
            u, zsum = _tc_mid(part, dis, zsum)
        else:
            full = _tc_final(part, dis, zsum, W, b.reshape(1, _D))

    sr3 = (
        jnp.concatenate([senders, receivers])
        .astype(jnp.int32)
        .reshape(_NW, 2, 128)
    )
    both = _sc_take(full, sr3)
    nb = senders.shape[0]
    return both[:nb], both[nb:]


# speedup vs baseline: 1.3298x; 1.0353x over previous
import functools

import jax
import jax.numpy as jnp
from jax import lax
from jax.experimental import pallas as pl
from jax.experimental.pallas import tpu as pltpu
from jax.experimental.pallas import tpu_sc as plsc

_NC = 2
_NS = 16
_NW = _NC * _NS
_D = 128
_EC = 128
_R0 = 128
_R1 = 32


def _sc_mesh():
    return plsc.VectorSubcoreMesh(core_axis_name="c", subcore_axis_name="s")


def _sc_degree(dst2, n_pad):
    rows_per_tile = dst2.shape[0] // _NW
    npt = n_pad // _NS

    @functools.partial(
        pl.kernel,
        out_type=[
            jax.ShapeDtypeStruct((n_pad,), jnp.float32),
            jax.ShapeDtypeStruct((n_pad,), jnp.float32),
        ],
        mesh=_sc_mesh(),
        scratch_types=[
            pltpu.VMEM((rows_per_tile, _EC), jnp.int32),
            pltpu.VMEM((_EC,), jnp.float32),
            pltpu.VMEM((npt,), jnp.float32),
            pltpu.VMEM_SHARED((n_pad,), jnp.float32),
            pltpu.SemaphoreType.DMA,
        ],
    )
    def k(dst_hbm, out0, out1, dst_v, ones_v, stage_v, deg_sh, sem):
        c = lax.axis_index("c")
        s = lax.axis_index("s")
        w = c * _NS + s

        def fill_ones(i, _):
            ones_v[pl.ds(i * 16, 16)] = jnp.ones((16,), jnp.float32)
            return 0

        lax.fori_loop(0, _EC // 16, fill_ones, 0)

        def fill_zero(i, _):
            stage_v[pl.ds(i * 16, 16)] = jnp.zeros((16,), jnp.float32)
            return 0

        lax.fori_loop(0, npt // 16, fill_zero, 0)
        pltpu.sync_copy(stage_v, deg_sh.at[pl.ds(s * npt, npt)])
        pltpu.async_copy(
            dst_hbm.at[pl.ds(w * rows_per_tile, rows_per_tile)], dst_v, sem
        ).wait()
        plsc.subcore_barrier()

        def body(k, _):
            for j in range(4):
                pltpu.async_copy(
                    ones_v, deg_sh.at[dst_v.at[k * 4 + j]], sem, add=True
                )
            for j in range(4):
                pltpu.make_async_copy(
                    ones_v, deg_sh.at[pl.ds(0, _EC)], sem
                ).wait()
            return 0

        lax.fori_loop(0, rows_per_tile // 4, body, 0)
        plsc.subcore_barrier()
        pltpu.sync_copy(deg_sh.at[pl.ds(s * npt, npt)], stage_v)

        @pl.when(c == 0)
        def _():
            pltpu.sync_copy(stage_v, out0.at[pl.ds(s * npt, npt)])

        @pl.when(c == 1)
        def _():
            pltpu.sync_copy(stage_v, out1.at[pl.ds(s * npt, npt)])

    return k(dst2)


def _sc_propagate(u, src2, dst2, n_pad, r0, r1):
    npt = n_pad // _NS
    rmax = max(r0, r1)

    @functools.partial(
        pl.kernel,
        out_type=jax.ShapeDtypeStruct((_NC, n_pad, _D), jnp.float32),
        mesh=_sc_mesh(),
        scratch_types=[
            pltpu.VMEM((rmax, _EC), jnp.int32),
            pltpu.VMEM((rmax, _EC), jnp.int32),
            pltpu.VMEM_SHARED((n_pad, _D), jnp.float32),
            pltpu.VMEM((_EC, _D), jnp.float32),
            pltpu.SemaphoreType.DMA,
            pltpu.SemaphoreType.DMA,
        ],
    )
    def k(u_hbm, src_hbm, dst_hbm, out_hbm, src_v, dst_v, y_sh, rows_v,
          gsem, sem):
        c = lax.axis_index("c")
        s = lax.axis_index("s")
        rpt = jnp.where(c == 0, r0, r1)
        base_row = c * _NS * r0 + s * rpt

        def zrow(i, _):
            for kk in range(_D // 16):
                rows_v[i, pl.ds(kk * 16, 16)] = jnp.zeros((16,), jnp.float32)
            return 0

        lax.fori_loop(0, _EC, zrow, 0)

        def zsh(j, _):
            pltpu.sync_copy(rows_v, y_sh.at[pl.ds(s * npt + j * _EC, _EC)])
            return 0

        lax.fori_loop(0, npt // _EC, zsh, 0)
        pltpu.async_copy(src_hbm.at[pl.ds(base_row, rmax)], src_v, sem).wait()
        pltpu.async_copy(dst_hbm.at[pl.ds(base_row, rmax)], dst_v, sem).wait()
        plsc.subcore_barrier()

        pltpu.async_copy(u_hbm.at[src_v.at[0]], rows_v, gsem)

        def body(g, _):
            pltpu.make_async_copy(
                u_hbm.at[pl.ds(0, _EC)], rows_v, gsem
            ).wait()
            pltpu.sync_copy(rows_v, y_sh.at[dst_v.at[g]], add=True)

            @pl.when(g < rpt - 1)
            def _():
                pltpu.async_copy(u_hbm.at[src_v.at[g + 1]], rows_v, gsem)

            return 0

        lax.fori_loop(0, rpt, body, 0)
        plsc.subcore_barrier()

        pltpu.sync_copy(
            y_sh.at[pl.ds(s * npt, npt)], out_hbm.at[c, pl.ds(s * npt, npt)]
        )

    return k(u, src2, dst2)


def _sc_take(full, sr3):
    @functools.partial(
        pl.kernel,
        out_type=jax.ShapeDtypeStruct((_NW * 256, _D), jnp.float32),
        mesh=_sc_mesh(),
        scratch_types=[
            pltpu.VMEM((2, 128), jnp.int32),
            pltpu.VMEM((128, _D), jnp.float32),
            pltpu.SemaphoreType.DMA,
        ],
    )
    def k(full_hbm, sr_hbm, out_hbm, idx_v, rows_v, sem):
        c = lax.axis_index("c")
        s = lax.axis_index("s")
        w = c * _NS + s
        pltpu.async_copy(sr_hbm.at[w], idx_v, sem).wait()
        for j in range(2):
            pltpu.async_copy(full_hbm.at[idx_v.at[j]], rows_v, sem).wait()
            pltpu.sync_copy(rows_v, out_hbm.at[pl.ds(w * 256 + j * 128, 128)])

    return k(full, sr3)


def _tc_dis(d0, d1):
    def body(a_ref, b_ref, o_ref):
        deg = a_ref[...] + b_ref[...]
        o_ref[...] = jnp.where(deg > 0.0, lax.rsqrt(jnp.maximum(deg, 1.0)), 0.0)

    return pl.pallas_call(
        body, out_shape=jax.ShapeDtypeStruct(d0.shape, jnp.float32)
    )(d0, d1)


def _tc_prolog(emb_pad, dis):
    n_pad = emb_pad.shape[0]
    blk = n_pad // 8

    def body(emb_ref, dis_ref, u0_ref, z0_ref):
        x = emb_ref[...]
        u0_ref[...] = x * dis_ref[...]
        nrm = jnp.sqrt(jnp.sum(x * x, axis=1, keepdims=True))
        z0_ref[...] = x / jnp.maximum(nrm, 1e-12)

    return pl.pallas_call(
        body,
        grid=(8,),
        in_specs=[
            pl.BlockSpec((blk, _D), lambda i: (i, 0)),
            pl.BlockSpec((blk, 1), lambda i: (i, 0)),
        ],
        out_specs=[
            pl.BlockSpec((blk, _D), lambda i: (i, 0)),
            pl.BlockSpec((blk, _D), lambda i: (i, 0)),
        ],
        out_shape=[
            jax.ShapeDtypeStruct((n_pad, _D), jnp.float32),
            jax.ShapeDtypeStruct((n_pad, _D), jnp.float32),
        ],
    )(emb_pad, dis)


def _tc_mid(part, dis, zsum):
    n_pad = dis.shape[0]
    blk = n_pad // 8

    def body(part_ref, dis_ref, zsum_ref, u_ref, zout_ref):
        x = (part_ref[0] + part_ref[1]) * dis_ref[...]
        nrm = jnp.sqrt(jnp.sum(x * x, axis=1, keepdims=True))
        zout_ref[...] = zsum_ref[...] + x / jnp.maximum(nrm, 1e-12)
        u_ref[...] = x * dis_ref[...]

    return pl.pallas_call(
        body,
        grid=(8,),
        in_specs=[
            pl.BlockSpec((_NC, blk, _D), lambda i: (0, i, 0)),
            pl.BlockSpec((blk, 1), lambda i: (i, 0)),
            pl.BlockSpec((blk, _D), lambda i: (i, 0)),
        ],
        out_specs=[
            pl.BlockSpec((blk, _D), lambda i: (i, 0)),
            pl.BlockSpec((blk, _D), lambda i: (i, 0)),
        ],
        out_shape=[
            jax.ShapeDtypeStruct((n_pad, _D), jnp.float32),
            jax.ShapeDtypeStruct((n_pad, _D), jnp.float32),
        ],
    )(part, dis, zsum)


def _tc_final(part, dis, zsum, W, b2):
    n_pad = dis.shape[0]
    blk = n_pad // 8

    def body(part_ref, dis_ref, zsum_ref, w_ref, b_ref, out_ref):
        x = (part_ref[0] + part_ref[1]) * dis_ref[...]
        nrm = jnp.sqrt(jnp.sum(x * x, axis=1, keepdims=True))
        zm = (zsum_ref[...] + x / jnp.maximum(nrm, 1e-12)) * 0.25
        out_ref[...] = (
            lax.dot_general(
                zm,
                w_ref[...],
                (((1,), (1,)), ((), ())),
                preferred_element_type=jnp.float32,
            )
            + b_ref[...]
        )

    return pl.pallas_call(
        body,
        grid=(8,),
        in_specs=[
            pl.BlockSpec((_NC, blk, _D), lambda i: (0, i, 0)),
            pl.BlockSpec((blk, 1), lambda i: (i, 0)),
            pl.BlockSpec((blk, _D), lambda i: (i, 0)),
            pl.BlockSpec((_D, _D), lambda i: (0, 0)),
            pl.BlockSpec((1, _D), lambda i: (0, 0)),
        ],
        out_specs=pl.BlockSpec((blk, _D), lambda i: (i, 0)),
        out_shape=jax.ShapeDtypeStruct((n_pad, _D), jnp.float32),
    )(part, dis, zsum, W, b2)


def kernel(senders, receivers, emb, edge_index, W, b):
    n = emb.shape[0]
    n_pad = -(-n // 1280) * 1280
    e_rows = edge_index.shape[1] // _EC
    assert _NS * (_R0 + _R1) >= e_rows and max(_R0, _R1) % 8 == 0
    e_rows_pad = -(-(_NS * (_R0 + _R1) + max(_R0, _R1)) // 256) * 256
    src2 = edge_index[0].astype(jnp.int32).reshape(e_rows, _EC)
    dst2 = edge_index[1].astype(jnp.int32).reshape(e_rows, _EC)
    pad_rows = ((0, e_rows_pad - e_rows), (0, 0))
    src2 = jnp.pad(src2, pad_rows, constant_values=n)
    dst2 = jnp.pad(dst2, pad_rows, constant_values=n)
    emb_pad = jnp.pad(emb, ((0, n_pad - n), (0, 0)))

    deg0, deg1 = _sc_degree(dst2, n_pad)
    dis2d = _tc_dis(deg0.reshape(-1, 128), deg1.reshape(-1, 128))
    dis = dis2d.reshape(n_pad, 1)
    u, zsum = _tc_prolog(emb_pad, dis)
    for layer in range(3):
        part = _sc_propagate(u, src2, dst2, n_pad, _R0, _R1)
        if layer < 2:
            u, zsum = _tc_mid(part, dis, zsum)
        else:
            full = _tc_final(part, dis, zsum, W, b.reshape(1, _D))

    sr3 = (
        jnp.concatenate([senders, receivers])
        .astype(jnp.int32)
        .reshape(_NW, 2, 128)
    )
    both = _sc_take(full, sr3)
    nb = senders.shape[0]
    return both[:nb], both[nb:]
